# baseline (speedup 1.0000x reference)
import functools

import jax
import jax.numpy as jnp
from jax import lax
from jax.experimental import pallas as pl
from jax.experimental.pallas import tpu as pltpu
from jax.experimental.pallas import tpu_sc as plsc

VOCAB = 1000000
EMBED = 64
BATCH = 16384
X_MAX = 100.0
ALPHA = 0.75

NC = 2
NS = 16
NW = NC * NS
BPW = BATCH // NW
CHUNK = 128
NCHUNK = BPW // CHUNK
L = 16

_LN2 = 0.6931471805599453
_LN_XMAX = 4.605170185988092
_SQRT2 = 1.4142135623730951


def _vlog(x):
    bits = lax.bitcast_convert_type(x, jnp.int32)
    e = (bits >> 23) - 127
    m = lax.bitcast_convert_type((bits & 0x007FFFFF) | 0x3F800000, jnp.float32)
    big = m > _SQRT2
    e = jnp.where(big, e + 1, e)
    m = jnp.where(big, m * 0.5, m)
    t = (m - 1.0) / (m + 1.0)
    t2 = t * t
    poly = 2.0 * t * (1.0 + t2 * (1.0 / 3.0 + t2 * (0.2 + t2 * (1.0 / 7.0))))
    return e.astype(jnp.float32) * _LN2 + poly


def _glove_body(femb, cemb, fbias, cbias, cnt, fidx, cidx, out_hbm,
                idxf_v, idxc_v, frows, crows, fb_v, cb_v, cnt_v,
                w_v, lc_v, out_v, sem):
    wid = lax.axis_index("s") * NC + lax.axis_index("c")
    base = wid * BPW

    for i in range(NCHUNK):
        pltpu.sync_copy(fidx.at[pl.ds(base + i * CHUNK, CHUNK)], idxf_v.at[i])
        pltpu.sync_copy(cidx.at[pl.ds(base + i * CHUNK, CHUNK)], idxc_v.at[i])
    pltpu.sync_copy(cnt.at[pl.ds(base, BPW)], cnt_v)

    copies = []
    for i in range(NCHUNK):
        sl = pl.ds(i * CHUNK, CHUNK)
        copies.append(pltpu.async_copy(femb.at[idxf_v.at[i]], frows.at[sl], sem))
        copies.append(pltpu.async_copy(cemb.at[idxc_v.at[i]], crows.at[sl], sem))
        copies.append(pltpu.async_copy(fbias.at[idxf_v.at[i]], fb_v.at[sl], sem))
        copies.append(pltpu.async_copy(cbias.at[idxc_v.at[i]], cb_v.at[sl], sem))

    def wl_body(g, carry):
        sl = pl.ds(g * L, L)
        c = cnt_v[sl]
        lc = _vlog(c)
        w = jnp.exp(ALPHA * (lc - _LN_XMAX))
        w = jnp.minimum(w, 1.0)
        lc_v[sl] = lc
        w_v[sl] = w
        return carry

    lax.fori_loop(0, BPW // L, wl_body, 0)

    for c in copies:
        c.wait()

    lanes = lax.iota(jnp.int32, L)
    perms = [lanes ^ sh for sh in (1, 2, 4, 8)]

    def group_body(g, lossvec):
        sl = pl.ds(g * L, L)
        s16 = fb_v[sl] + cb_v[sl] + lc_v[sl]
        w16 = w_v[sl]
        d_vec = jnp.zeros((L,), jnp.float32)
        for k in range(L):
            b = g * L + k
            fr = frows.at[b]
            cr = crows.at[b]
            p = fr[pl.ds(0, L)] * cr[pl.ds(0, L)]
            for j in range(1, EMBED // L):
                p = p + fr[pl.ds(j * L, L)] * cr[pl.ds(j * L, L)]
            for perm in perms:
                p = p + jnp.take(p, perm)
            d_vec = jnp.where(lanes == k, p, d_vec)
        expr = d_vec + s16
        return lossvec + w16 * (expr * expr)

    lossvec = lax.fori_loop(0, BPW // L, group_body, jnp.zeros((L,), jnp.float32))

    out_v[...] = lossvec
    pltpu.sync_copy(out_v, out_hbm.at[wid])


@functools.partial(
    pl.kernel,
    out_type=jax.ShapeDtypeStruct((NW, L), jnp.float32),
    mesh=plsc.VectorSubcoreMesh(
        core_axis_name="c", subcore_axis_name="s", num_cores=NC, num_subcores=NS
    ),
    compiler_params=pltpu.CompilerParams(use_tc_tiling_on_sc=False),
    scratch_types=[
        pltpu.VMEM((NCHUNK, CHUNK), jnp.int32),
        pltpu.VMEM((NCHUNK, CHUNK), jnp.int32),
        pltpu.VMEM((BPW, EMBED), jnp.float32),
        pltpu.VMEM((BPW, EMBED), jnp.float32),
        pltpu.VMEM((BPW,), jnp.float32),
        pltpu.VMEM((BPW,), jnp.float32),
        pltpu.VMEM((BPW,), jnp.float32),
        pltpu.VMEM((BPW,), jnp.float32),
        pltpu.VMEM((BPW,), jnp.float32),
        pltpu.VMEM((L,), jnp.float32),
        pltpu.SemaphoreType.DMA,
    ],
)
def _glove_sc(femb, cemb, fbias, cbias, cnt, fidx, cidx, out_hbm, *scratch):
    _glove_body(femb, cemb, fbias, cbias, cnt, fidx, cidx, out_hbm, *scratch)


def kernel(focal_embeddings, context_embeddings, focal_biases, context_biases,
           coocurrence_count, focal_input, context_input):
    partials = _glove_sc(
        focal_embeddings,
        context_embeddings,
        focal_biases,
        context_biases,
        coocurrence_count,
        focal_input.astype(jnp.int32),
        context_input.astype(jnp.int32),
    )
    return jnp.sum(partials)
